# baseline (speedup 1.0000x reference)
import functools

import jax
import jax.numpy as jnp
from jax import lax
from jax.experimental import pallas as pl
from jax.experimental.pallas import tpu as pltpu
from jax.experimental.pallas import tpu_sc as plsc

N = 100000
D = 128
NW = 32
C = 200
NCHUNK = N // C
ITERS = (NCHUNK + NW - 1) // NW
FULL = NCHUNK - (ITERS - 1) * NW
KC = 3
KO = 2

_mesh = plsc.VectorSubcoreMesh(core_axis_name="c", subcore_axis_name="s")


@functools.partial(
    pl.kernel,
    out_type=jax.ShapeDtypeStruct((N, 2 * D), jnp.float32),
    mesh=_mesh,
    scratch_types=(
        [pltpu.VMEM((C,), jnp.int32)] * KC
        + [pltpu.VMEM((C, D), jnp.float32)] * KC
        + [pltpu.VMEM_SHARED((16, KO, C, D), jnp.float32)]
        + [pltpu.VMEM((16,), jnp.int32)]
        + [pltpu.SemaphoreType.DMA] * (1 + 2 * KC + 2 * KO)
    ),
)
def _sc_gather_concat(ctx_hbm, obj_hbm, idx_hbm, ev_hbm, out_hbm, *scr):
    idx_bufs = scr[:KC]
    p = KC
    ctx_v = scr[p:p + KC]; p += KC
    obj_s = scr[p]; p += 1
    ev_v = scr[p]; p += 1
    sem_idx = scr[p]; p += 1
    sem_g = scr[p:p + KC]; p += KC
    sem_wg = scr[p:p + KC]; p += KC
    sem_o = scr[p:p + KO]; p += KO
    sem_wo = scr[p:p + KO]; p += KO
    sid = lax.axis_index("s")

    wid = lax.axis_index("s") * 2 + lax.axis_index("c")
    last_ok = wid < FULL

    pltpu.sync_copy(ev_hbm, ev_v)
    shuffled = ev_v[...][0] != 0

    def rows(i):
        return pl.ds((wid + i * NW) * C, C)

    def guarded(i, fn):
        if i == ITERS - 1:
            pl.when(last_ok)(fn)
        else:
            fn()

    def pipeline(make_ctx_in):
        ctx_in = [None] * ITERS
        ctx_out = [None] * ITERS
        obj_in = [None] * ITERS
        obj_out = [None] * ITERS

        def finish(j):
            bc, bo = j % KC, j % KO
            ctx_out[j] = pltpu.make_async_copy(
                ctx_v[bc], out_hbm.at[rows(j), pl.ds(0, D)], sem_wg[bc])
            obj_out[j] = pltpu.make_async_copy(
                obj_s.at[sid, bo], out_hbm.at[rows(j), pl.ds(D, D)],
                sem_wo[bo])
            guarded(j, ctx_in[j].wait)
            guarded(j, ctx_out[j].start)
            guarded(j, obj_in[j].wait)
            guarded(j, obj_out[j].start)

        for i in range(ITERS):
            if i >= KC:
                guarded(i - KC, ctx_out[i - KC].wait)
            if i >= KO:
                guarded(i - KO, obj_out[i - KO].wait)
            bc, bo = i % KC, i % KO
            ctx_in[i] = make_ctx_in(i, ctx_v[bc], sem_g[bc])
            obj_in[i] = pltpu.make_async_copy(
                obj_hbm.at[rows(i)], obj_s.at[sid, bo], sem_o[bo])
            guarded(i, ctx_in[i].start)
            guarded(i, obj_in[i].start)
            if i >= 1:
                finish(i - 1)
        finish(ITERS - 1)
        for j in range(max(0, ITERS - KC), ITERS):
            guarded(j, ctx_out[j].wait)
        for j in range(max(0, ITERS - KO), ITERS):
            guarded(j, obj_out[j].wait)

    @pl.when(jnp.logical_not(shuffled))
    def _identity_path():
        pipeline(lambda i, dst, sem: pltpu.make_async_copy(
            ctx_hbm.at[rows(i)], dst, sem))

    @pl.when(shuffled)
    def _gather_path():
        def gather_in(i, dst, sem):
            b = i % KC
            idd = pltpu.make_async_copy(
                idx_hbm.at[rows(i)], idx_bufs[b], sem_idx)
            guarded(i, idd.start)
            guarded(i, idd.wait)
            return pltpu.make_async_copy(ctx_hbm.at[idx_bufs[b]], dst, sem)

        pipeline(gather_in)


def kernel(context_output, object_output, eval_random):
    num = context_output.shape[0]
    with jax.ensure_compile_time_eval():
        perm_idx = jnp.asarray(
            jax.random.permutation(jax.random.key(42), num), jnp.int32)
        identity_idx = jnp.arange(num, dtype=jnp.int32)
    random_idx = jnp.where(eval_random, perm_idx, identity_idx)
    ev = jnp.broadcast_to(jnp.asarray(eval_random, jnp.int32), (16,))
    return _sc_gather_concat(context_output, object_output, random_idx, ev)
